# baseline (speedup 1.0000x reference)
import dataclasses
import functools

import jax
import jax.numpy as jnp
from jax import lax
from jax.experimental import pallas as pl
from jax.experimental.pallas import tpu as pltpu
from jax.experimental.pallas import tpu_sc as plsc

NC = 2
NS = 16
NW = NC * NS
CHUNK = 96
DEPTH = 3

_CP = pltpu.CompilerParams()
if "needs_layout_passes" in pltpu.CompilerParams.__dataclass_fields__:
    _CP = dataclasses.replace(_CP, needs_layout_passes=False)


def _make_deg_kernel(n_pad, e_pad):
    ept = e_pad // NW
    nch = ept // CHUNK
    mesh = plsc.VectorSubcoreMesh(core_axis_name="c", subcore_axis_name="s")

    @functools.partial(
        pl.kernel,
        mesh=mesh,
        out_type=jax.ShapeDtypeStruct((NW, 1, n_pad), jnp.float32),
        compiler_params=_CP,
        scratch_types=[
            pltpu.VMEM((nch, CHUNK), jnp.int32),
            pltpu.VMEM((n_pad,), jnp.float32),
        ],
    )
    def deg_kernel(dst_hbm, out_hbm, idx_v, hist):
        cid = lax.axis_index("c")
        sid = lax.axis_index("s")
        wid = cid * NS + sid
        pltpu.sync_copy(dst_hbm.at[wid], idx_v)
        zero = jnp.zeros((16,), jnp.float32)

        @pl.loop(0, n_pad, step=16)
        def _(i):
            hist[pl.ds(i, 16)] = zero

        ones = jnp.ones((16,), jnp.float32)

        @pl.loop(0, nch)
        def _(j):
            for k in range(CHUNK // 16):
                idx = idx_v[j, pl.ds(k * 16, 16)]
                plsc.addupdate_scatter(hist, [idx], ones)

        pltpu.sync_copy(hist, out_hbm.at[wid, 0])

    return deg_kernel


def _make_msg_kernel(n_pad, e_pad, d):
    ept = e_pad // NW
    nch = ept // CHUNK
    rows_pc = n_pad // NS
    mesh = plsc.VectorSubcoreMesh(core_axis_name="c", subcore_axis_name="s")

    @functools.partial(
        pl.kernel,
        mesh=mesh,
        out_type=jax.ShapeDtypeStruct((NC, n_pad, d), jnp.float32),
        scratch_types=[
            pltpu.VMEM((DEPTH, 2, CHUNK), jnp.int32),
            pltpu.VMEM((DEPTH, CHUNK, d), jnp.float32),
            pltpu.VMEM_SHARED((n_pad, d), jnp.float32),
        ] + [pltpu.SemaphoreType.DMA] * (2 * DEPTH),
    )
    def msg_kernel(y_hbm, sd_hbm, zeros_hbm, out_hbm,
                   sd_v, rows_v, accum, *sems):
        cid = lax.axis_index("c")
        sid = lax.axis_index("s")
        wid = cid * NS + sid
        pltpu.sync_copy(zeros_hbm, accum.at[pl.ds(sid * rows_pc, rows_pc)])
        plsc.subcore_barrier()

        si = list(sems[:DEPTH])
        sg = list(sems[DEPTH:])
        hi = [None] * DEPTH
        hg = [None] * DEPTH

        def idx_load(j):
            b = j % DEPTH
            hi[b] = pltpu.async_copy(sd_hbm.at[wid, j], sd_v.at[b], si[b])

        def gather(j):
            b = j % DEPTH
            hg[b] = pltpu.async_copy(
                y_hbm.at[sd_v.at[b, 0]], rows_v.at[b], sg[b])

        for j in range(min(DEPTH, nch)):
            idx_load(j)
        for j in range(min(DEPTH - 1, nch)):
            hi[j].wait()
            gather(j)
        for j in range(nch):
            b = j % DEPTH
            if j + DEPTH - 1 < nch:
                nb = (j + DEPTH - 1) % DEPTH
                hi[nb].wait()
                gather(j + DEPTH - 1)
            hg[b].wait()
            pltpu.sync_copy(rows_v.at[b], accum.at[sd_v.at[b, 1]], add=True)
            if j + DEPTH < nch:
                idx_load(j + DEPTH)

        plsc.subcore_barrier()
        pltpu.sync_copy(
            accum.at[pl.ds(sid * rows_pc, rows_pc)],
            out_hbm.at[cid, pl.ds(sid * rows_pc, rows_pc)],
        )

    return msg_kernel


def _tc_layer1(degp_t, x_pad, w):
    n_pad = x_pad.shape[0]
    d = w.shape[1]

    def body(degp_ref, x_ref, w_ref, dinv_ref, y_ref):
        xw = jnp.dot(x_ref[...], w_ref[...],
                     preferred_element_type=jnp.float32)
        deg = jnp.sum(degp_ref[...], axis=1, keepdims=True) + 1.0
        dinv = lax.rsqrt(deg)
        dinv_ref[...] = dinv
        y_ref[...] = xw * dinv

    return pl.pallas_call(
        body,
        out_shape=(
            jax.ShapeDtypeStruct((n_pad, 1), jnp.float32),
            jax.ShapeDtypeStruct((n_pad, d), jnp.float32),
        ),
    )(degp_t, x_pad, w)


def _tc_layer2(parts, y, dinv, b, w):
    n_pad = y.shape[0]
    d_out = w.shape[1]

    def body(p_ref, y_ref, dinv_ref, b_ref, w_ref, y2_ref):
        s = p_ref[0] + p_ref[1] + y_ref[...]
        h = jnp.maximum(s * dinv_ref[...] + b_ref[...], 0.0)
        y2_ref[...] = jnp.dot(h, w_ref[...],
                              preferred_element_type=jnp.float32) * dinv_ref[...]

    return pl.pallas_call(
        body,
        out_shape=jax.ShapeDtypeStruct((n_pad, d_out), jnp.float32),
    )(parts, y, dinv, b, w)


def _tc_finish(parts, y, dinv, b, batch2d, g):
    n_pad, d = y.shape
    d_out = b.shape[1]

    def body(p_ref, y_ref, dinv_ref, b_ref, batch_ref, o_ref):
        s = (p_ref[0] + p_ref[1] + y_ref[...])[:, :d_out]
        h = jnp.maximum(s * dinv_ref[...] + b_ref[...], 0.0)
        gids = lax.broadcasted_iota(jnp.int32, (g, n_pad), 0)
        mask = (gids == batch_ref[...]).astype(jnp.float32)
        o_ref[...] = jnp.dot(mask, h, preferred_element_type=jnp.float32)

    return pl.pallas_call(
        body,
        out_shape=jax.ShapeDtypeStruct((g, d_out), jnp.float32),
    )(parts, y, dinv, b, batch2d)


def kernel(x, edge_index, batch, W1, b1, W2, b2):
    n, d_in = x.shape
    e = edge_index.shape[1]
    d_hid = W1.shape[1]
    d_out = W2.shape[1]
    g = 16

    n_pad = ((n + 1 + 127) // 128) * 128
    e_chunk = NW * CHUNK
    e_pad = ((e + e_chunk - 1) // e_chunk) * e_chunk
    ept = e_pad // NW
    nch = ept // CHUNK
    rows_pc = n_pad // NS

    pad_e = jnp.arange(e_pad - e, dtype=jnp.int32)
    src = jnp.concatenate(
        [edge_index[0], pad_e % n]
    ).reshape(NW, nch, CHUNK)
    dst = jnp.concatenate(
        [edge_index[1], n + pad_e % (n_pad - n)]
    ).reshape(NW, nch, CHUNK)
    x_pad = jnp.pad(x, ((0, n_pad - n), (0, 0)))
    batch2d = jnp.pad(batch, (0, n_pad - n),
                      constant_values=g).reshape(1, n_pad)
    zeros_m1 = jnp.zeros((rows_pc, d_hid), jnp.float32)
    w2_pad = jnp.pad(W2, ((0, 0), (0, d_hid - d_out)))

    sd = jnp.stack([src, dst], axis=2)

    degp = _make_deg_kernel(n_pad, e_pad)(dst)
    degp_t = jnp.transpose(degp.reshape(NW, n_pad))

    dinv, y1 = _tc_layer1(degp_t, x_pad, W1)
    parts1 = _make_msg_kernel(n_pad, e_pad, d_hid)(y1, sd, zeros_m1)
    y2 = _tc_layer2(parts1, y1, dinv, b1.reshape(1, d_hid), w2_pad)
    parts2 = _make_msg_kernel(n_pad, e_pad, d_hid)(y2, sd, zeros_m1)
    return _tc_finish(parts2, y2, dinv, b2.reshape(1, d_out), batch2d, g)


# speedup vs baseline: 1.2435x; 1.2435x over previous
import dataclasses
import functools

import jax
import jax.numpy as jnp
from jax import lax
from jax.experimental import pallas as pl
from jax.experimental.pallas import tpu as pltpu
from jax.experimental.pallas import tpu_sc as plsc

NC = 2
NS = 16
NW = NC * NS
CHUNK = 112
DEPTH = 3
SDEPTH = 5

_CP = pltpu.CompilerParams()
if "needs_layout_passes" in pltpu.CompilerParams.__dataclass_fields__:
    _CP = dataclasses.replace(_CP, needs_layout_passes=False)


def _make_deg_kernel(n_pad, e_pad):
    ept = e_pad // NW
    nch = ept // CHUNK
    mesh = plsc.VectorSubcoreMesh(core_axis_name="c", subcore_axis_name="s")

    @functools.partial(
        pl.kernel,
        mesh=mesh,
        out_type=jax.ShapeDtypeStruct((NW, 1, n_pad), jnp.float32),
        compiler_params=_CP,
        scratch_types=[
            pltpu.VMEM((nch, CHUNK), jnp.int32),
            pltpu.VMEM((n_pad,), jnp.float32),
        ],
    )
    def deg_kernel(dst_hbm, out_hbm, idx_v, hist):
        cid = lax.axis_index("c")
        sid = lax.axis_index("s")
        wid = cid * NS + sid
        pltpu.sync_copy(dst_hbm.at[wid], idx_v)
        zero = jnp.zeros((16,), jnp.float32)

        @pl.loop(0, n_pad, step=16)
        def _(i):
            hist[pl.ds(i, 16)] = zero

        ones = jnp.ones((16,), jnp.float32)

        @pl.loop(0, nch)
        def _(j):
            for k in range(CHUNK // 16):
                idx = idx_v[j, pl.ds(k * 16, 16)]
                plsc.addupdate_scatter(hist, [idx], ones)

        pltpu.sync_copy(hist, out_hbm.at[wid, 0])

    return deg_kernel


def _make_msg_kernel(n_pad, e_pad, d):
    ept = e_pad // NW
    nch = ept // CHUNK
    rows_pc = n_pad // NS
    mesh = plsc.VectorSubcoreMesh(core_axis_name="c", subcore_axis_name="s")

    @functools.partial(
        pl.kernel,
        mesh=mesh,
        out_type=jax.ShapeDtypeStruct((NC, n_pad, d), jnp.float32),
        scratch_types=[
            pltpu.VMEM((SDEPTH, 2, CHUNK), jnp.int32),
            pltpu.VMEM((DEPTH, CHUNK, d), jnp.float32),
            pltpu.VMEM_SHARED((n_pad, d), jnp.float32),
        ] + [pltpu.SemaphoreType.DMA] * (SDEPTH + 2 * DEPTH),
    )
    def msg_kernel(y_hbm, sd_hbm, zeros_hbm, out_hbm,
                   sd_v, rows_v, accum, *sems):
        cid = lax.axis_index("c")
        sid = lax.axis_index("s")
        wid = cid * NS + sid
        pltpu.sync_copy(zeros_hbm, accum.at[pl.ds(sid * rows_pc, rows_pc)])
        plsc.subcore_barrier()

        si = list(sems[:SDEPTH])
        sg = list(sems[SDEPTH:SDEPTH + DEPTH])
        ss = list(sems[SDEPTH + DEPTH:])
        hi = [None] * SDEPTH
        hg = [None] * DEPTH
        hs = [None] * DEPTH

        def idx_load(j):
            b = j % SDEPTH
            hi[b] = pltpu.async_copy(sd_hbm.at[wid, j], sd_v.at[b], si[b])

        def gather(j):
            hg[j % DEPTH] = pltpu.async_copy(
                y_hbm.at[sd_v.at[j % SDEPTH, 0]], rows_v.at[j % DEPTH],
                sg[j % DEPTH])

        def scatter(j):
            hs[j % DEPTH] = pltpu.async_copy(
                rows_v.at[j % DEPTH], accum.at[sd_v.at[j % SDEPTH, 1]],
                ss[j % DEPTH], add=True)

        for k in range(min(DEPTH, nch)):
            idx_load(k)
        for k in range(min(DEPTH - 1, nch)):
            hi[k % SDEPTH].wait()
            gather(k)
        for j in range(nch):
            rb = j % DEPTH
            if j + DEPTH - 1 < nch:
                if j >= 1:
                    hs[(j + DEPTH - 1) % DEPTH].wait()
                hi[(j + DEPTH - 1) % SDEPTH].wait()
                gather(j + DEPTH - 1)
            hg[rb].wait()
            scatter(j)
            if j + DEPTH < nch:
                idx_load(j + DEPTH)
        for k in range(max(0, nch - DEPTH), nch):
            hs[k % DEPTH].wait()

        plsc.subcore_barrier()
        pltpu.sync_copy(
            accum.at[pl.ds(sid * rows_pc, rows_pc)],
            out_hbm.at[cid, pl.ds(sid * rows_pc, rows_pc)],
        )

    return msg_kernel


def _tc_layer1(degp_t, x_pad, w):
    n_pad = x_pad.shape[0]
    d = w.shape[1]

    def body(degp_ref, x_ref, w_ref, dinv_ref, y_ref):
        xw = jnp.dot(x_ref[...], w_ref[...],
                     preferred_element_type=jnp.float32)
        deg = jnp.sum(degp_ref[...], axis=1, keepdims=True) + 1.0
        dinv = lax.rsqrt(deg)
        dinv_ref[...] = dinv
        y_ref[...] = xw * dinv

    return pl.pallas_call(
        body,
        out_shape=(
            jax.ShapeDtypeStruct((n_pad, 1), jnp.float32),
            jax.ShapeDtypeStruct((n_pad, d), jnp.float32),
        ),
    )(degp_t, x_pad, w)


def _tc_layer2(parts, y, dinv, b, w):
    n_pad = y.shape[0]
    d_out = w.shape[1]

    def body(p_ref, y_ref, dinv_ref, b_ref, w_ref, y2_ref):
        s = p_ref[0] + p_ref[1] + y_ref[...]
        h = jnp.maximum(s * dinv_ref[...] + b_ref[...], 0.0)
        y2_ref[...] = jnp.dot(h, w_ref[...],
                              preferred_element_type=jnp.float32) * dinv_ref[...]

    return pl.pallas_call(
        body,
        out_shape=jax.ShapeDtypeStruct((n_pad, d_out), jnp.float32),
    )(parts, y, dinv, b, w)


def _tc_finish(parts, y, dinv, b, batch2d, g):
    n_pad, d = y.shape
    d_out = b.shape[1]

    def body(p_ref, y_ref, dinv_ref, b_ref, batch_ref, o_ref):
        s = (p_ref[0] + p_ref[1] + y_ref[...])[:, :d_out]
        h = jnp.maximum(s * dinv_ref[...] + b_ref[...], 0.0)
        gids = lax.broadcasted_iota(jnp.int32, (g, n_pad), 0)
        mask = (gids == batch_ref[...]).astype(jnp.float32)
        o_ref[...] = jnp.dot(mask, h, preferred_element_type=jnp.float32)

    return pl.pallas_call(
        body,
        out_shape=jax.ShapeDtypeStruct((g, d_out), jnp.float32),
    )(parts, y, dinv, b, batch2d)


def kernel(x, edge_index, batch, W1, b1, W2, b2):
    n, d_in = x.shape
    e = edge_index.shape[1]
    d_hid = W1.shape[1]
    d_out = W2.shape[1]
    g = 16

    n_pad = ((n + 1 + 127) // 128) * 128
    e_chunk = NW * CHUNK
    e_pad = ((e + e_chunk - 1) // e_chunk) * e_chunk
    ept = e_pad // NW
    nch = ept // CHUNK
    rows_pc = n_pad // NS

    pad_e = jnp.arange(e_pad - e, dtype=jnp.int32)
    src = jnp.concatenate(
        [edge_index[0], pad_e % n]
    ).reshape(NW, nch, CHUNK)
    dst = jnp.concatenate(
        [edge_index[1], n + pad_e % (n_pad - n)]
    ).reshape(NW, nch, CHUNK)
    x_pad = jnp.pad(x, ((0, n_pad - n), (0, 0)))
    batch2d = jnp.pad(batch, (0, n_pad - n),
                      constant_values=g).reshape(1, n_pad)
    zeros_m1 = jnp.zeros((rows_pc, d_hid), jnp.float32)
    w2_pad = jnp.pad(W2, ((0, 0), (0, d_hid - d_out)))

    sd = jnp.stack([src, dst], axis=2)

    degp = _make_deg_kernel(n_pad, e_pad)(dst)
    degp_t = jnp.transpose(degp.reshape(NW, n_pad))

    dinv, y1 = _tc_layer1(degp_t, x_pad, W1)
    parts1 = _make_msg_kernel(n_pad, e_pad, d_hid)(y1, sd, zeros_m1)
    y2 = _tc_layer2(parts1, y1, dinv, b1.reshape(1, d_hid), w2_pad)
    parts2 = _make_msg_kernel(n_pad, e_pad, d_hid)(y2, sd, zeros_m1)
    return _tc_finish(parts2, y2, dinv, b2.reshape(1, d_out), batch2d, g)
